# Initial kernel scaffold; baseline (speedup 1.0000x reference)
import jax
import jax.numpy as jnp
from jax.experimental import pallas as pl


def kernel(features, edge_index, W1, b1, W2, b2):
    raise NotImplementedError("write your pallas kernel here")


# speedup vs baseline: 6.9854x; 6.9854x over previous
import functools

import jax
import jax.numpy as jnp
from jax import lax
from jax.experimental import pallas as pl
from jax.experimental.pallas import tpu as pltpu
from jax.experimental.pallas import tpu_sc as plsc

_NC = 2
_NS = 16
_NW = _NC * _NS
_DEGW = 16


def _make_seg_sum(N, E, D, with_deg):
  EPW = E // _NW
  C = 80
  NCHUNK = EPW // C
  RPT = N // _NS
  CP = 125
  NCP = RPT // CP
  assert EPW * _NW == E and NCHUNK * C == EPW and NCP * CP == RPT

  out_type = [jax.ShapeDtypeStruct((_NC, N, D), jnp.float32)]
  scratch = [
      pltpu.VMEM((C,), jnp.int32),
      pltpu.VMEM((C,), jnp.int32),
      pltpu.VMEM((C, D), jnp.float32),
      pltpu.VMEM((CP, D), jnp.float32),
      pltpu.VMEM_SHARED((N, D), jnp.float32),
      pltpu.SemaphoreType.DMA,
  ]
  if with_deg:
    out_type.append(jax.ShapeDtypeStruct((_NC, N, _DEGW), jnp.float32))
    scratch += [
        pltpu.VMEM((C, _DEGW), jnp.float32),
        pltpu.VMEM((CP, _DEGW), jnp.float32),
        pltpu.VMEM_SHARED((N, _DEGW), jnp.float32),
    ]

  mesh = plsc.VectorSubcoreMesh(core_axis_name="c", subcore_axis_name="s")

  @functools.partial(
      pl.kernel, out_type=tuple(out_type), mesh=mesh,
      scratch_types=tuple(scratch),
      compiler_params=pltpu.CompilerParams(use_tc_tiling_on_sc=False))
  def seg(y_hbm, src_hbm, dst_hbm, *refs):
    if with_deg:
      (agg_out, deg_out, srcv, dstv, rows, zb, acc, sem, ones, zd,
       dacc) = refs
    else:
      (agg_out, srcv, dstv, rows, zb, acc, sem) = refs
    c = lax.axis_index("c")
    s = lax.axis_index("s")
    wid = c * _NS + s
    base = wid * EPW

    def zrow(j, _):
      for t in range(D // 16):
        zb[j, pl.ds(16 * t, 16)] = jnp.zeros((16,), jnp.float32)
      return 0
    lax.fori_loop(0, CP, zrow, 0)
    if with_deg:
      def onesrow(j, _):
        for t in range(_DEGW // 16):
          ones[j, pl.ds(16 * t, 16)] = jnp.full((16,), 1.0, jnp.float32)
        return 0
      lax.fori_loop(0, C, onesrow, 0)
      def zdrow(j, _):
        for t in range(_DEGW // 16):
          zd[j, pl.ds(16 * t, 16)] = jnp.zeros((16,), jnp.float32)
        return 0
      lax.fori_loop(0, CP, zdrow, 0)

    r0 = s * RPT
    for k in range(NCP):
      pltpu.sync_copy(zb, acc.at[pl.ds(r0 + k * CP, CP)])
      if with_deg:
        pltpu.sync_copy(zd, dacc.at[pl.ds(r0 + k * CP, CP)])
    plsc.subcore_barrier()

    def step(i, _):
      off = base + i * C
      pltpu.sync_copy(src_hbm.at[pl.ds(off, C)], srcv)
      pltpu.sync_copy(dst_hbm.at[pl.ds(off, C)], dstv)
      pltpu.async_copy(y_hbm.at[srcv], rows, sem).wait()
      pltpu.sync_copy(rows, acc.at[dstv], add=True)
      if with_deg:
        pltpu.sync_copy(ones, dacc.at[dstv], add=True)
      return 0
    lax.fori_loop(0, NCHUNK, step, 0)
    plsc.subcore_barrier()

    pltpu.sync_copy(acc.at[pl.ds(r0, RPT)], agg_out.at[c, pl.ds(r0, RPT)])
    if with_deg:
      pltpu.sync_copy(dacc.at[pl.ds(r0, RPT)], deg_out.at[c, pl.ds(r0, RPT)])

  return seg


def _mm_body(x_ref, w_ref, o_ref):
  o_ref[...] = jnp.dot(x_ref[...], w_ref[...],
                       preferred_element_type=jnp.float32)


def _mid_body(aggp_ref, degp_ref, b1_ref, w2_ref, y2_ref, degi_ref):
  agg = aggp_ref[0] + aggp_ref[1]
  deg = degp_ref[0] + degp_ref[1]
  degi = 1.0 / jnp.maximum(deg, 1.0)
  h = agg * jnp.concatenate([degi, degi], axis=1) + b1_ref[...]
  h = jnp.maximum(h, 0.0)
  y2_ref[...] = jnp.dot(h, w2_ref[...], preferred_element_type=jnp.float32)
  degi_ref[...] = degi


def _fin_body(aggp_ref, degi_ref, b2_ref, o_ref):
  agg = aggp_ref[0] + aggp_ref[1]
  res = jnp.maximum(agg * degi_ref[...] + b2_ref[...], 0.0)
  o_ref[...] = res[:, :o_ref.shape[1]]


def kernel(features, edge_index, W1, b1, W2, b2):
  N, _ = features.shape
  E = edge_index.shape[1]
  D_HID = W1.shape[1]
  D_OUT = W2.shape[1]
  src = edge_index[0]
  dst = edge_index[1]

  y1 = pl.pallas_call(
      _mm_body,
      out_shape=jax.ShapeDtypeStruct((N, D_HID), jnp.float32),
  )(features, W1)

  aggp, degp = _make_seg_sum(N, E, D_HID, True)(y1, src, dst)

  W2p = jnp.zeros((D_HID, 16), jnp.float32).at[:, :D_OUT].set(W2)
  y2, degi = pl.pallas_call(
      _mid_body,
      out_shape=(jax.ShapeDtypeStruct((N, 16), jnp.float32),
                 jax.ShapeDtypeStruct((N, 16), jnp.float32)),
  )(aggp, degp, b1.reshape(1, D_HID), W2p)

  agg2p = _make_seg_sum(N, E, 16, False)(y2, src, dst)
  if isinstance(agg2p, (tuple, list)):
    agg2p = agg2p[0]

  b2p = jnp.zeros((1, 16), jnp.float32).at[0, :D_OUT].set(b2)
  out = pl.pallas_call(
      _fin_body,
      out_shape=jax.ShapeDtypeStruct((N, D_OUT), jnp.float32),
  )(agg2p, degi, b2p)
  return out


# speedup vs baseline: 11.3486x; 1.6246x over previous
import functools

import jax
import jax.numpy as jnp
from jax import lax
from jax.experimental import pallas as pl
from jax.experimental.pallas import tpu as pltpu
from jax.experimental.pallas import tpu_sc as plsc

_NC = 2
_NS = 16
_NW = _NC * _NS
_DEGW = 16


def _make_seg_sum(N, E, D, with_deg):
  EPW = E // _NW
  C = 80
  NCHUNK = EPW // C
  RPT = N // _NS
  CP = 125
  NCP = RPT // CP
  assert EPW * _NW == E and NCHUNK * C == EPW and NCP * CP == RPT

  out_type = [jax.ShapeDtypeStruct((_NC, N, D), jnp.float32)]
  scratch = [
      pltpu.VMEM((NCHUNK, C), jnp.int32),
      pltpu.VMEM((NCHUNK, C), jnp.int32),
      pltpu.VMEM((C, D), jnp.float32),
      pltpu.VMEM((CP, D), jnp.float32),
      pltpu.VMEM_SHARED((N, D), jnp.float32),
      pltpu.SemaphoreType.DMA,
  ]
  if with_deg:
    out_type.append(jax.ShapeDtypeStruct((_NC, N, _DEGW), jnp.float32))
    scratch += [
        pltpu.VMEM((C, _DEGW), jnp.float32),
        pltpu.VMEM((CP, _DEGW), jnp.float32),
        pltpu.VMEM_SHARED((N, _DEGW), jnp.float32),
    ]

  mesh = plsc.VectorSubcoreMesh(core_axis_name="c", subcore_axis_name="s")

  @functools.partial(
      pl.kernel, out_type=tuple(out_type), mesh=mesh,
      scratch_types=tuple(scratch),
      compiler_params=pltpu.CompilerParams(use_tc_tiling_on_sc=False))
  def seg(y_hbm, src_hbm, dst_hbm, *refs):
    if with_deg:
      (agg_out, deg_out, srcv, dstv, rows, zb, acc, sem, ones, zd,
       dacc) = refs
    else:
      (agg_out, srcv, dstv, rows, zb, acc, sem) = refs
    c = lax.axis_index("c")
    s = lax.axis_index("s")
    wid = c * _NS + s

    pltpu.sync_copy(src_hbm.at[pl.ds(wid * NCHUNK, NCHUNK)], srcv)
    pltpu.sync_copy(dst_hbm.at[pl.ds(wid * NCHUNK, NCHUNK)], dstv)

    def zrow(j, _):
      for t in range(D // 16):
        zb[j, pl.ds(16 * t, 16)] = jnp.zeros((16,), jnp.float32)
      return 0
    lax.fori_loop(0, CP, zrow, 0)
    if with_deg:
      def onesrow(j, _):
        for t in range(_DEGW // 16):
          ones[j, pl.ds(16 * t, 16)] = jnp.full((16,), 1.0, jnp.float32)
        return 0
      lax.fori_loop(0, C, onesrow, 0)
      def zdrow(j, _):
        for t in range(_DEGW // 16):
          zd[j, pl.ds(16 * t, 16)] = jnp.zeros((16,), jnp.float32)
        return 0
      lax.fori_loop(0, CP, zdrow, 0)

    r0 = s * RPT
    for k in range(NCP):
      pltpu.sync_copy(zb, acc.at[pl.ds(r0 + k * CP, CP)])
      if with_deg:
        pltpu.sync_copy(zd, dacc.at[pl.ds(r0 + k * CP, CP)])
    plsc.subcore_barrier()

    def step(i, _):
      pltpu.async_copy(y_hbm.at[srcv.at[i]], rows, sem).wait()
      pltpu.sync_copy(rows, acc.at[dstv.at[i]], add=True)
      if with_deg:
        pltpu.sync_copy(ones, dacc.at[dstv.at[i]], add=True)
      return 0
    lax.fori_loop(0, NCHUNK, step, 0)
    plsc.subcore_barrier()

    pltpu.sync_copy(acc.at[pl.ds(r0, RPT)], agg_out.at[c, pl.ds(r0, RPT)])
    if with_deg:
      pltpu.sync_copy(dacc.at[pl.ds(r0, RPT)], deg_out.at[c, pl.ds(r0, RPT)])

  def run(y, src, dst):
    return seg(y, src.reshape(E // C, C), dst.reshape(E // C, C))

  return run


def _mm_body(x_ref, w_ref, o_ref):
  o_ref[...] = jnp.dot(x_ref[...], w_ref[...],
                       preferred_element_type=jnp.float32)


def _mid_body(aggp_ref, degp_ref, b1_ref, w2_ref, y2_ref, degi_ref):
  agg = aggp_ref[0] + aggp_ref[1]
  deg = degp_ref[0] + degp_ref[1]
  degi = 1.0 / jnp.maximum(deg, 1.0)
  h = agg * jnp.concatenate([degi, degi], axis=1) + b1_ref[...]
  h = jnp.maximum(h, 0.0)
  y2_ref[...] = jnp.dot(h, w2_ref[...], preferred_element_type=jnp.float32)
  degi_ref[...] = degi


def _fin_body(aggp_ref, degi_ref, b2_ref, o_ref):
  agg = aggp_ref[0] + aggp_ref[1]
  res = jnp.maximum(agg * degi_ref[...] + b2_ref[...], 0.0)
  o_ref[...] = res[:, :o_ref.shape[1]]


def kernel(features, edge_index, W1, b1, W2, b2):
  N, _ = features.shape
  E = edge_index.shape[1]
  D_HID = W1.shape[1]
  D_OUT = W2.shape[1]
  src = edge_index[0]
  dst = edge_index[1]

  y1 = pl.pallas_call(
      _mm_body,
      out_shape=jax.ShapeDtypeStruct((N, D_HID), jnp.float32),
  )(features, W1)

  aggp, degp = _make_seg_sum(N, E, D_HID, True)(y1, src, dst)

  W2p = jnp.zeros((D_HID, 16), jnp.float32).at[:, :D_OUT].set(W2)
  y2, degi = pl.pallas_call(
      _mid_body,
      out_shape=(jax.ShapeDtypeStruct((N, 16), jnp.float32),
                 jax.ShapeDtypeStruct((N, 16), jnp.float32)),
  )(aggp, degp, b1.reshape(1, D_HID), W2p)

  agg2p = _make_seg_sum(N, E, 16, False)(y2, src, dst)
  if isinstance(agg2p, (tuple, list)):
    agg2p = agg2p[0]

  b2p = jnp.zeros((1, 16), jnp.float32).at[0, :D_OUT].set(b2)
  out = pl.pallas_call(
      _fin_body,
      out_shape=jax.ShapeDtypeStruct((N, D_OUT), jnp.float32),
  )(agg2p, degi, b2p)
  return out


# speedup vs baseline: 16.8002x; 1.4804x over previous
import functools

import jax
import jax.numpy as jnp
from jax import lax
from jax.experimental import pallas as pl
from jax.experimental.pallas import tpu as pltpu
from jax.experimental.pallas import tpu_sc as plsc

_NC = 2
_NS = 16
_NW = _NC * _NS
_DEGW = 16


def _make_seg_sum(N, E, D, with_deg):
  EPW = E // _NW
  C = 80
  NCHUNK = EPW // C
  RPT = N // _NS
  CP = 125
  NCP = RPT // CP
  assert EPW * _NW == E and NCHUNK * C == EPW and NCP * CP == RPT

  out_type = [jax.ShapeDtypeStruct((_NC, N, D), jnp.float32)]
  scratch = [
      pltpu.VMEM((NCHUNK, C), jnp.int32),
      pltpu.VMEM((NCHUNK, C), jnp.int32),
      pltpu.VMEM((C, D), jnp.float32),
      pltpu.VMEM((C, D), jnp.float32),
      pltpu.VMEM((CP, D), jnp.float32),
      pltpu.VMEM_SHARED((N, D), jnp.float32),
      pltpu.SemaphoreType.DMA,
      pltpu.SemaphoreType.DMA,
  ]
  if with_deg:
    out_type.append(jax.ShapeDtypeStruct((_NC, N, _DEGW), jnp.float32))
    scratch += [
        pltpu.VMEM((C, _DEGW), jnp.float32),
        pltpu.VMEM((CP, _DEGW), jnp.float32),
        pltpu.VMEM_SHARED((N, _DEGW), jnp.float32),
    ]

  mesh = plsc.VectorSubcoreMesh(core_axis_name="c", subcore_axis_name="s")

  @functools.partial(
      pl.kernel, out_type=tuple(out_type), mesh=mesh,
      scratch_types=tuple(scratch),
      compiler_params=pltpu.CompilerParams(use_tc_tiling_on_sc=False))
  def seg(y_hbm, src_hbm, dst_hbm, *refs):
    if with_deg:
      (agg_out, deg_out, srcv, dstv, rows_a, rows_b, zb, acc, sema, semb,
       ones, zd, dacc) = refs
    else:
      (agg_out, srcv, dstv, rows_a, rows_b, zb, acc, sema, semb) = refs
    c = lax.axis_index("c")
    s = lax.axis_index("s")
    wid = c * _NS + s

    pltpu.sync_copy(src_hbm.at[pl.ds(wid * NCHUNK, NCHUNK)], srcv)
    pltpu.sync_copy(dst_hbm.at[pl.ds(wid * NCHUNK, NCHUNK)], dstv)

    def zrow(j, _):
      for t in range(D // 16):
        zb[j, pl.ds(16 * t, 16)] = jnp.zeros((16,), jnp.float32)
      return 0
    lax.fori_loop(0, CP, zrow, 0)
    if with_deg:
      def onesrow(j, _):
        for t in range(_DEGW // 16):
          ones[j, pl.ds(16 * t, 16)] = jnp.full((16,), 1.0, jnp.float32)
        return 0
      lax.fori_loop(0, C, onesrow, 0)
      def zdrow(j, _):
        for t in range(_DEGW // 16):
          zd[j, pl.ds(16 * t, 16)] = jnp.zeros((16,), jnp.float32)
        return 0
      lax.fori_loop(0, CP, zdrow, 0)

    r0 = s * RPT
    for k in range(NCP):
      pltpu.sync_copy(zb, acc.at[pl.ds(r0 + k * CP, CP)])
      if with_deg:
        pltpu.sync_copy(zd, dacc.at[pl.ds(r0 + k * CP, CP)])
    plsc.subcore_barrier()

    def fire(ch, buf, sm):
      pltpu.async_copy(y_hbm.at[srcv.at[ch]], buf, sm)

    def drain(buf, sm):
      pltpu.make_async_copy(y_hbm.at[srcv.at[0]], buf, sm).wait()

    def scat(ch, buf):
      pltpu.sync_copy(buf, acc.at[dstv.at[ch]], add=True)
      if with_deg:
        pltpu.sync_copy(ones, dacc.at[dstv.at[ch]], add=True)

    assert NCHUNK % 2 == 1
    fire(0, rows_a, sema)
    def step(i, _):
      fire(2 * i + 1, rows_b, semb)
      drain(rows_a, sema)
      scat(2 * i, rows_a)
      fire(2 * i + 2, rows_a, sema)
      drain(rows_b, semb)
      scat(2 * i + 1, rows_b)
      return 0
    lax.fori_loop(0, (NCHUNK - 1) // 2, step, 0)
    drain(rows_a, sema)
    scat(NCHUNK - 1, rows_a)
    plsc.subcore_barrier()

    pltpu.sync_copy(acc.at[pl.ds(r0, RPT)], agg_out.at[c, pl.ds(r0, RPT)])
    if with_deg:
      pltpu.sync_copy(dacc.at[pl.ds(r0, RPT)], deg_out.at[c, pl.ds(r0, RPT)])

  def run(y, src, dst):
    return seg(y, src.reshape(E // C, C), dst.reshape(E // C, C))

  return run


def _mm_body(x_ref, w_ref, o_ref):
  o_ref[...] = jnp.dot(x_ref[...], w_ref[...],
                       preferred_element_type=jnp.float32)


def _mid_body(aggp_ref, degp_ref, b1_ref, w2_ref, y2_ref, degi_ref):
  agg = aggp_ref[0] + aggp_ref[1]
  deg = degp_ref[0] + degp_ref[1]
  degi = 1.0 / jnp.maximum(deg, 1.0)
  h = agg * jnp.concatenate([degi, degi], axis=1) + b1_ref[...]
  h = jnp.maximum(h, 0.0)
  y2_ref[...] = jnp.dot(h, w2_ref[...], preferred_element_type=jnp.float32)
  degi_ref[...] = degi


def _fin_body(aggp_ref, degi_ref, b2_ref, o_ref):
  agg = aggp_ref[0] + aggp_ref[1]
  res = jnp.maximum(agg * degi_ref[...] + b2_ref[...], 0.0)
  o_ref[...] = res[:, :o_ref.shape[1]]


def kernel(features, edge_index, W1, b1, W2, b2):
  N, _ = features.shape
  E = edge_index.shape[1]
  D_HID = W1.shape[1]
  D_OUT = W2.shape[1]
  src = edge_index[0]
  dst = edge_index[1]

  y1 = pl.pallas_call(
      _mm_body,
      out_shape=jax.ShapeDtypeStruct((N, D_HID), jnp.float32),
  )(features, W1)

  aggp, degp = _make_seg_sum(N, E, D_HID, True)(y1, src, dst)

  W2p = jnp.zeros((D_HID, 16), jnp.float32).at[:, :D_OUT].set(W2)
  y2, degi = pl.pallas_call(
      _mid_body,
      out_shape=(jax.ShapeDtypeStruct((N, 16), jnp.float32),
                 jax.ShapeDtypeStruct((N, 16), jnp.float32)),
  )(aggp, degp, b1.reshape(1, D_HID), W2p)

  agg2p = _make_seg_sum(N, E, 16, False)(y2, src, dst)
  if isinstance(agg2p, (tuple, list)):
    agg2p = agg2p[0]

  b2p = jnp.zeros((1, 16), jnp.float32).at[0, :D_OUT].set(b2)
  out = pl.pallas_call(
      _fin_body,
      out_shape=jax.ShapeDtypeStruct((N, D_OUT), jnp.float32),
  )(agg2p, degi, b2p)
  return out
